# baseline (speedup 1.0000x reference)
import functools

import jax
import jax.numpy as jnp
from jax import lax
from jax.experimental import pallas as pl
from jax.experimental.pallas import tpu as pltpu
from jax.experimental.pallas import tpu_sc as plsc

B, A, F = 256, 128, 128
D = 16
DB = 16
C = 32
N = B * A

NC, NS = 2, 16
NW = NC * NS
ROWS_PER_W = N // NW
Q = 128
NQ = ROWS_PER_W // Q
STEP = 16
NSTEP = Q // STEP
ACC_ROWS = Q + 1


def _sc_neigh_sum(atoms_flat, edges_1d):
    mesh = plsc.VectorSubcoreMesh(
        core_axis_name="c", subcore_axis_name="s",
        num_cores=NC, num_subcores=NS)

    @functools.partial(
        pl.kernel,
        out_type=jax.ShapeDtypeStruct((N, F), jnp.float32),
        mesh=mesh,
        scratch_types=[
            pltpu.VMEM((Q * D,), jnp.int32),
            pltpu.VMEM((Q * D,), jnp.int32),
            pltpu.VMEM((STEP * D,), jnp.int32),
            pltpu.VMEM((STEP * D,), jnp.int32),
            pltpu.VMEM((STEP * D,), jnp.int32),
            pltpu.VMEM((STEP * D,), jnp.int32),
            pltpu.VMEM((2, STEP * D, F), jnp.float32),
            pltpu.VMEM_SHARED((NS * ACC_ROWS, F), jnp.float32),
            pltpu.VMEM_SHARED((NS * 2 * A, F), jnp.float32),
            pltpu.SemaphoreType.DMA,
            pltpu.SemaphoreType.DMA,
            pltpu.SemaphoreType.DMA,
            pltpu.SemaphoreType.DMA,
            pltpu.SemaphoreType.DMA,
            pltpu.SemaphoreType.DMA,
        ],
    )
    def k(atoms_hbm, edges_hbm, out_hbm, eb0, eb1, si0, si1, di0, di1,
          gbuf, acc, mol, sg0, sg1, se0, se1, sm0, sm1):
        cid = lax.axis_index("c")
        sid = lax.axis_index("s")
        wid = cid * NS + sid
        reg_base = sid * ACC_ROWS
        ebufs = (eb0, eb1)
        src_idx = (si0, si1)
        dst_idx = (di0, di1)
        gsems = (sg0, sg1)
        esems = (se0, se1)
        msems = (sm0, sm1)

        def mol_base(reg):
            return (sid * 2 + reg) * A

        def stage(q, reg):
            base = wid * ROWS_PER_W + q * Q
            e = pltpu.make_async_copy(
                edges_hbm.at[pl.ds(base * D, Q * D)], ebufs[reg], esems[reg])
            m = pltpu.make_async_copy(
                atoms_hbm.at[pl.ds(base, Q)],
                mol.at[pl.ds(mol_base(reg), Q)], msems[reg])
            return e, m

        def compute_idx(reg, step, buf):
            soff = step * (STEP * D)
            for j in range(STEP):
                e = ebufs[reg][pl.ds(soff + j * D, D)]
                src_idx[buf][pl.ds(j * D, D)] = (
                    jnp.maximum(e, 0) + mol_base(reg))
                dst_idx[buf][pl.ds(j * D, D)] = jnp.where(
                    e >= 0, reg_base + step * STEP + j, reg_base + Q)

        def gather(buf):
            return pltpu.make_async_copy(
                mol.at[src_idx[buf]], gbuf.at[buf], gsems[buf])

        e0, m0 = stage(0, 0)
        e0.start()
        m0.start()
        for q in range(NQ):
            reg = q % 2
            base = wid * ROWS_PER_W + q * Q
            ew, mw = stage(q, reg)
            ew.wait()
            mw.wait()
            if q + 1 < NQ:
                en, mn = stage(q + 1, 1 - reg)
                en.start()
                mn.start()
            pltpu.sync_copy(atoms_hbm.at[pl.ds(base, Q)],
                            acc.at[pl.ds(reg_base, Q)])
            compute_idx(reg, 0, 0)
            gather(0).start()

            def pair(i, carry):
                for kk in range(2):
                    s = 2 * i + kk
                    buf, nbuf = kk, 1 - kk
                    if kk == 0:
                        compute_idx(reg, s + 1, nbuf)
                        gather(nbuf).start()
                    else:
                        @pl.when(s + 1 < NSTEP)
                        def _():
                            compute_idx(reg, s + 1, nbuf)
                            gather(nbuf).start()
                    gather(buf).wait()
                    pltpu.sync_copy(gbuf.at[buf], acc.at[dst_idx[buf]],
                                    add=True)
                return carry

            lax.fori_loop(0, NSTEP // 2, pair, 0)
            pltpu.sync_copy(acc.at[pl.ds(reg_base, Q)],
                            out_hbm.at[pl.ds(base, Q)])

    return k(atoms_flat, edges_1d)


ROWS_TC = 512
GRID = N // ROWS_TC


def _tc_body(x_ref, bonds_ref, edges_ref, wa_ref, wb_ref, b_ref,
             sel_ref, o_ref):
    x = x_ref[...]
    h = jnp.dot(x, wa_ref[...], preferred_element_type=jnp.float32)
    bsum = jnp.sum(bonds_ref[...], axis=1)
    h = h + jnp.dot(bsum, wb_ref[...], preferred_element_type=jnp.float32)
    h = jnp.maximum(h + b_ref[...], 0.0)
    validf = (edges_ref[...] != -1).astype(jnp.float32)
    degf = jnp.sum(validf, axis=1, keepdims=True)
    dpat = (jax.lax.broadcasted_iota(jnp.int32, (1, D * C), 1) // C
            ).astype(jnp.float32)
    mask = (degf == dpat).astype(jnp.float32)
    o_ref[...] = jnp.dot(h * mask, sel_ref[...],
                         preferred_element_type=jnp.float32)


def _tc_dense(summed, bonds2d, edges_flat, wa, wb_big, b_all, sel):
    return pl.pallas_call(
        _tc_body,
        grid=(GRID,),
        in_specs=[
            pl.BlockSpec((ROWS_TC, F), lambda i: (i, 0)),
            pl.BlockSpec((ROWS_TC, D, DB), lambda i: (i, 0, 0)),
            pl.BlockSpec((ROWS_TC, D), lambda i: (i, 0)),
            pl.BlockSpec((F, D * C), lambda i: (0, 0)),
            pl.BlockSpec((DB, D * C), lambda i: (0, 0)),
            pl.BlockSpec((1, D * C), lambda i: (0, 0)),
            pl.BlockSpec((D * C, C), lambda i: (0, 0)),
        ],
        out_specs=pl.BlockSpec((ROWS_TC, C), lambda i: (i, 0)),
        out_shape=jax.ShapeDtypeStruct((N, C), jnp.float32),
    )(summed, bonds2d, edges_flat, wa, wb_big, b_all, sel)


def kernel(atoms, bonds, edges, W, b):
    atoms_flat = atoms.reshape(N, F)
    edges_flat = edges.reshape(N, D)
    bonds3d = bonds.reshape(N, D, DB)
    w_all = jnp.transpose(W, (1, 0, 2)).reshape(F + DB, D * C)
    wa = w_all[:F]
    wb = w_all[F:]
    b_all = b.reshape(1, D * C)
    sel = jnp.tile(jnp.eye(C, dtype=jnp.float32), (D, 1))
    summed = _sc_neigh_sum(atoms_flat, edges_flat.reshape(-1))
    out = _tc_dense(summed, bonds3d, edges_flat, wa, wb, b_all, sel)
    return out.reshape(B, A, C)


# speedup vs baseline: 1.0636x; 1.0636x over previous
import functools

import jax
import jax.numpy as jnp
from jax import lax
from jax.experimental import pallas as pl
from jax.experimental.pallas import tpu as pltpu
from jax.experimental.pallas import tpu_sc as plsc

B, A, F = 256, 128, 128
D = 16
DB = 16
C = 32
N = B * A

NC, NS = 2, 16
NW = NC * NS
ROWS_PER_W = N // NW
Q = 128
NQ = ROWS_PER_W // Q
STEP = 16
NSTEP = Q // STEP
ACC_ROWS = Q + 1


def _sc_neigh_sum(atoms_flat, edges_1d):
    mesh = plsc.VectorSubcoreMesh(
        core_axis_name="c", subcore_axis_name="s",
        num_cores=NC, num_subcores=NS)

    @functools.partial(
        pl.kernel,
        out_type=jax.ShapeDtypeStruct((N, F), jnp.float32),
        mesh=mesh,
        scratch_types=[
            pltpu.VMEM((Q * D,), jnp.int32),
            pltpu.VMEM((Q * D,), jnp.int32),
            pltpu.VMEM((STEP * D,), jnp.int32),
            pltpu.VMEM((STEP * D,), jnp.int32),
            pltpu.VMEM((STEP * D,), jnp.int32),
            pltpu.VMEM((STEP * D,), jnp.int32),
            pltpu.VMEM((2, STEP * D, F), jnp.float32),
            pltpu.VMEM_SHARED((NS * ACC_ROWS, F), jnp.float32),
            pltpu.VMEM_SHARED((NS * 2 * A, F), jnp.float32),
            pltpu.SemaphoreType.DMA,
            pltpu.SemaphoreType.DMA,
            pltpu.SemaphoreType.DMA,
            pltpu.SemaphoreType.DMA,
            pltpu.SemaphoreType.DMA,
            pltpu.SemaphoreType.DMA,
        ],
    )
    def k(atoms_hbm, edges_hbm, out_hbm, eb0, eb1, si0, si1, di0, di1,
          gbuf, acc, mol, sg0, sg1, se0, se1, sm0, sm1):
        cid = lax.axis_index("c")
        sid = lax.axis_index("s")
        wid = cid * NS + sid
        reg_base = sid * ACC_ROWS
        ebufs = (eb0, eb1)
        src_idx = (si0, si1)
        dst_idx = (di0, di1)
        gsems = (sg0, sg1)
        esems = (se0, se1)
        msems = (sm0, sm1)

        def mol_base(reg):
            return (sid * 2 + reg) * A

        def stage(q, reg):
            base = wid * ROWS_PER_W + q * Q
            e = pltpu.make_async_copy(
                edges_hbm.at[pl.ds(base * D, Q * D)], ebufs[reg], esems[reg])
            m = pltpu.make_async_copy(
                atoms_hbm.at[pl.ds(base, Q)],
                mol.at[pl.ds(mol_base(reg), Q)], msems[reg])
            return e, m

        def compute_idx(reg, step, buf):
            soff = step * (STEP * D)
            for j in range(STEP):
                e = ebufs[reg][pl.ds(soff + j * D, D)]
                src_idx[buf][pl.ds(j * D, D)] = (
                    jnp.maximum(e, 0) + mol_base(reg))
                dst_idx[buf][pl.ds(j * D, D)] = jnp.where(
                    e >= 0, reg_base + step * STEP + j, reg_base + Q)

        def gather(buf):
            return pltpu.make_async_copy(
                mol.at[src_idx[buf]], gbuf.at[buf], gsems[buf])

        e0, m0 = stage(0, 0)
        e0.start()
        m0.start()
        for q in range(NQ):
            reg = q % 2
            base = wid * ROWS_PER_W + q * Q
            ew, mw = stage(q, reg)
            ew.wait()
            mw.wait()
            if q + 1 < NQ:
                en, mn = stage(q + 1, 1 - reg)
                en.start()
                mn.start()
            pltpu.sync_copy(atoms_hbm.at[pl.ds(base, Q)],
                            acc.at[pl.ds(reg_base, Q)])
            compute_idx(reg, 0, 0)
            gather(0).start()

            def pair(i, carry):
                for kk in range(2):
                    s = 2 * i + kk
                    buf, nbuf = kk, 1 - kk
                    if kk == 0:
                        compute_idx(reg, s + 1, nbuf)
                        gather(nbuf).start()
                    else:
                        @pl.when(s + 1 < NSTEP)
                        def _():
                            compute_idx(reg, s + 1, nbuf)
                            gather(nbuf).start()
                    gather(buf).wait()
                    pltpu.sync_copy(gbuf.at[buf], acc.at[dst_idx[buf]],
                                    add=True)
                return carry

            lax.fori_loop(0, NSTEP // 2, pair, 0)
            pltpu.sync_copy(acc.at[pl.ds(reg_base, Q)],
                            out_hbm.at[pl.ds(base, Q)])

    return k(atoms_flat, edges_1d)


ROWS_TC = 512
GRID = N // ROWS_TC


def _tc_body(x_ref, bonds_ref, edges_ref, wa_ref, wb_ref, b_ref,
             sel_ref, o_ref):
    x = x_ref[...]
    h = jnp.dot(x, wa_ref[...], preferred_element_type=jnp.float32)
    h = h + jnp.dot(bonds_ref[...], wb_ref[...],
                    preferred_element_type=jnp.float32)
    h = jnp.maximum(h + b_ref[...], 0.0)
    validf = (edges_ref[...] != -1).astype(jnp.float32)
    degf = jnp.sum(validf, axis=1, keepdims=True)
    dpat = (jax.lax.broadcasted_iota(jnp.int32, (1, D * C), 1) // C
            ).astype(jnp.float32)
    mask = (degf == dpat).astype(jnp.float32)
    o_ref[...] = jnp.dot(h * mask, sel_ref[...],
                         preferred_element_type=jnp.float32)


def _tc_dense(summed, bonds2d, edges_flat, wa, wb_big, b_all, sel):
    return pl.pallas_call(
        _tc_body,
        grid=(GRID,),
        in_specs=[
            pl.BlockSpec((ROWS_TC, F), lambda i: (i, 0)),
            pl.BlockSpec((ROWS_TC, D * DB), lambda i: (i, 0)),
            pl.BlockSpec((ROWS_TC, D), lambda i: (i, 0)),
            pl.BlockSpec((F, D * C), lambda i: (0, 0)),
            pl.BlockSpec((D * DB, D * C), lambda i: (0, 0)),
            pl.BlockSpec((1, D * C), lambda i: (0, 0)),
            pl.BlockSpec((D * C, C), lambda i: (0, 0)),
        ],
        out_specs=pl.BlockSpec((ROWS_TC, C), lambda i: (i, 0)),
        out_shape=jax.ShapeDtypeStruct((N, C), jnp.float32),
    )(summed, bonds2d, edges_flat, wa, wb_big, b_all, sel)


def kernel(atoms, bonds, edges, W, b):
    atoms_flat = atoms.reshape(N, F)
    edges_flat = edges.reshape(N, D)
    bonds2d = bonds.reshape(N, D * DB)
    w_all = jnp.transpose(W, (1, 0, 2)).reshape(F + DB, D * C)
    wa = w_all[:F]
    wb_big = jnp.tile(w_all[F:], (D, 1))
    b_all = b.reshape(1, D * C)
    sel = jnp.tile(jnp.eye(C, dtype=jnp.float32), (D, 1))
    summed = _sc_neigh_sum(atoms_flat, edges_flat.reshape(-1))
    out = _tc_dense(summed, bonds2d, edges_flat, wa, wb_big, b_all, sel)
    return out.reshape(B, A, C)
